# baseline (speedup 1.0000x reference)
import functools

import jax
import jax.numpy as jnp
import numpy as np
from jax import lax
from jax.experimental import pallas as pl
from jax.experimental.pallas import tpu as pltpu
from jax.experimental.pallas import tpu_sc as plsc

N = 10000
E = 160000
D = 256
DH = 128
NC = 2
NS = 16
L = 16

CHUNK = 32
EPT = E // NS
NCHUNKS = (EPT - 16) // CHUNK
TAIL = EPT - NCHUNKS * CHUNK

_P32 = np.arange(32).reshape(2, 16).T.ravel()
_PERM = np.concatenate([g * 32 + _P32 for g in range(D // 32)])
ROWS_PT = 624
ROWS_TAIL = N - ROWS_PT * NS


def _proj_body(x_ref, wk_ref, wq_ref, wv_ref, ws_ref, bk_ref, bq_ref, bv_ref,
               bo_ref, k_ref, q_ref, v_ref, s_ref):
    x = x_ref[...]
    k = jnp.dot(x, wk_ref[...], preferred_element_type=jnp.float32) + bk_ref[...]
    q = jnp.dot(x, wq_ref[...], preferred_element_type=jnp.float32) + bq_ref[...]
    v = jnp.dot(x, wv_ref[...], preferred_element_type=jnp.float32) + bv_ref[...]
    k_ref[...] = k.astype(jnp.bfloat16)
    q_ref[...] = q.astype(jnp.bfloat16)
    v_ref[...] = v.astype(jnp.bfloat16)
    s_ref[...] = jnp.dot(x, ws_ref[...], preferred_element_type=jnp.float32) + bo_ref[...]


def _tc_proj(x, W_k, b_k, W_q, b_q, W_v, b_v, W_s, b_o):
    BLK = 1000
    grid = (N // BLK, NC)
    w_spec = pl.BlockSpec((D, DH), lambda i, j: (0, j))
    b_spec = pl.BlockSpec((1, DH), lambda i, j: (0, j))
    kqv_spec = pl.BlockSpec((BLK, DH), lambda i, j: (i, j))
    skip_spec = pl.BlockSpec((BLK, DH), lambda i, j: (j * (N // BLK) + i, 0))
    return pl.pallas_call(
        _proj_body,
        grid=grid,
        in_specs=[pl.BlockSpec((BLK, D), lambda i, j: (i, 0))] + [w_spec] * 4 + [b_spec] * 4,
        out_specs=[kqv_spec] * 3 + [skip_spec],
        out_shape=[jax.ShapeDtypeStruct((N, D), jnp.bfloat16)] * 3
        + [jax.ShapeDtypeStruct((NC * N, DH), jnp.float32)],
    )(x, W_k, W_q, W_v, W_s, b_k[None, :], b_q[None, :], b_v[None, :], b_o[None, :])


def _edge_body(xe_ref, w1_ref, w2_ref, b1_ref, b2_ref, e1_ref, e2_ref):
    xe = xe_ref[...]
    e1 = jnp.dot(xe, w1_ref[...], preferred_element_type=jnp.float32) + b1_ref[...]
    e2 = jnp.dot(xe, w2_ref[...], preferred_element_type=jnp.float32) + b2_ref[...]
    e1_ref[...] = e1.astype(jnp.bfloat16)
    e2_ref[...] = e2.astype(jnp.bfloat16)


def _tc_edge(x_edge, W_e1, b_e1, W_e2, b_e2):
    BLK = 4000
    DE = x_edge.shape[1]
    grid = (E // BLK, NC)
    w_spec = pl.BlockSpec((DE, DH), lambda i, j: (0, j))
    b_spec = pl.BlockSpec((1, DH), lambda i, j: (0, j))
    out_spec = pl.BlockSpec((BLK, DH), lambda i, j: (j * (E // BLK) + i, 0))
    out_shape = jax.ShapeDtypeStruct((NC * E, DH), jnp.bfloat16)
    return pl.pallas_call(
        _edge_body,
        grid=grid,
        in_specs=[pl.BlockSpec((BLK, DE), lambda i, j: (i, 0))] + [w_spec] * 2 + [b_spec] * 2,
        out_specs=[out_spec] * 2,
        out_shape=[out_shape] * 2,
    )(x_edge, W_e1, W_e2, b_e1[None, :], b_e2[None, :])


def _sc_body(k_hbm, q_hbm, v_hbm, ef_hbm, src_hbm, dst_hbm, skip_hbm, out_hbm,
             sidx, didx, dscat, tdsc,
             buf_k, buf_q, buf_v, buf_e, res,
             agg, sem_i, sem_g):
    c = lax.axis_index("c")
    s = lax.axis_index("s")

    row0 = s * ROWS_PT
    pltpu.sync_copy(skip_hbm.at[pl.ds(c * N + row0, ROWS_PT)],
                    agg.at[pl.ds(row0, ROWS_PT)])

    @pl.when(s == NS - 1)
    def _():
        pltpu.sync_copy(skip_hbm.at[pl.ds(c * N + NS * ROWS_PT, ROWS_TAIL)],
                        agg.at[pl.ds(NS * ROWS_PT, ROWS_TAIL)])

    plsc.subcore_barrier()

    ebase = s * EPT

    def clamp(i):
        return jnp.minimum(i, NCHUNKS - 1)

    def stage_idx(i, b):
        base = ebase + clamp(i) * CHUNK
        pltpu.async_copy(src_hbm.at[pl.ds(base, CHUNK)], sidx[b], sem_i[b])
        pltpu.async_copy(dst_hbm.at[pl.ds(base, CHUNK)], didx[b], sem_i[b])

    def wait_idx(b):
        pltpu.make_async_copy(src_hbm.at[pl.ds(ebase, CHUNK)], sidx[b], sem_i[b]).wait()
        pltpu.make_async_copy(dst_hbm.at[pl.ds(ebase, CHUNK)], didx[b], sem_i[b]).wait()

    def fire_gathers(i, b):
        for j in range(CHUNK // L):
            sl = pl.ds(j * L, L)
            dscat[b][sl] = didx[b][sl]
        base = ebase + clamp(i) * CHUNK
        pltpu.async_copy(k_hbm.at[didx[b]], buf_k[b], sem_g[b])
        pltpu.async_copy(q_hbm.at[sidx[b]], buf_q[b], sem_g[b])
        pltpu.async_copy(v_hbm.at[sidx[b]], buf_v[b], sem_g[b])
        pltpu.async_copy(ef_hbm.at[pl.ds(c * E + base, CHUNK)], buf_e[b], sem_g[b])

    def wait_gathers(b):
        pltpu.make_async_copy(k_hbm.at[didx[b]], buf_k[b], sem_g[b]).wait()
        pltpu.make_async_copy(q_hbm.at[sidx[b]], buf_q[b], sem_g[b]).wait()
        pltpu.make_async_copy(v_hbm.at[sidx[b]], buf_v[b], sem_g[b]).wait()
        pltpu.make_async_copy(ef_hbm.at[pl.ds(c * E, CHUNK)], buf_e[b], sem_g[b]).wait()

    def compute(b, nedge):
        cw = c * (DH // 2)

        def edge_body(r, carry2):
            for g in range(DH // 32):
                slw = pl.ds(cw + g * L, L)
                kk = plsc.bitcast(buf_k[b][r, slw], jnp.bfloat16)
                qq = plsc.bitcast(buf_q[b][r, slw], jnp.bfloat16)
                ee = plsc.bitcast(buf_e[b][r, pl.ds(g * L, L)], jnp.bfloat16)
                z = kk + qq + ee
                z_lo, z_hi = plsc.unpack(z, format=plsc.PackFormat.INTERLEAVED)
                vv = plsc.bitcast(buf_v[b][r, slw], jnp.bfloat16)
                v_lo, v_hi = plsc.unpack(vv, format=plsc.PackFormat.INTERLEAVED)
                g_lo = 1.0 / (1.0 + jnp.exp(-z_lo))
                g_hi = 1.0 / (1.0 + jnp.exp(-z_hi))
                res[r, pl.ds(g * 32, L)] = g_lo * v_lo
                res[r, pl.ds(g * 32 + L, L)] = g_hi * v_hi
            return carry2

        lax.fori_loop(0, nedge, edge_body, 0)

    stage_idx(0, 0)
    stage_idx(1, 1)
    wait_idx(0)
    fire_gathers(0, 0)

    def pipe_body(ig, carry):
        for b in (0, 1):
            i = 2 * ig + b
            wait_gathers(b)
            stage_idx(i + 2, b)
            wait_idx(1 - b)
            fire_gathers(i + 1, 1 - b)
            compute(b, CHUNK)
            pltpu.sync_copy(res, agg.at[dscat[b]], add=True)
        return carry

    lax.fori_loop(0, NCHUNKS // 2, pipe_body, 0)

    wait_gathers(0)
    wait_idx(1)

    tbase = ebase + NCHUNKS * CHUNK
    pltpu.sync_copy(src_hbm.at[pl.ds(tbase, TAIL)], sidx[0].at[pl.ds(0, TAIL)])
    pltpu.sync_copy(dst_hbm.at[pl.ds(tbase, TAIL)], didx[0].at[pl.ds(0, TAIL)])
    tdsc[...] = didx[0][pl.ds(0, L)]
    tsidx = sidx[0].at[pl.ds(0, TAIL)]
    cp_k = pltpu.async_copy(k_hbm.at[tdsc], buf_k[0].at[pl.ds(0, TAIL)], sem_g[0])
    cp_q = pltpu.async_copy(q_hbm.at[tsidx], buf_q[0].at[pl.ds(0, TAIL)], sem_g[0])
    cp_v = pltpu.async_copy(v_hbm.at[tsidx], buf_v[0].at[pl.ds(0, TAIL)], sem_g[0])
    cp_e = pltpu.async_copy(ef_hbm.at[pl.ds(c * E + tbase, TAIL)],
                            buf_e[0].at[pl.ds(0, TAIL)], sem_g[0])
    cp_k.wait()
    cp_q.wait()
    cp_v.wait()
    cp_e.wait()
    compute(0, TAIL)
    pltpu.sync_copy(res.at[pl.ds(0, TAIL)], agg.at[tdsc], add=True)

    plsc.subcore_barrier()

    pltpu.sync_copy(agg.at[pl.ds(row0, ROWS_PT)],
                    out_hbm.at[pl.ds(c * N + row0, ROWS_PT)])

    @pl.when(s == NS - 1)
    def _():
        pltpu.sync_copy(agg.at[pl.ds(NS * ROWS_PT, ROWS_TAIL)],
                        out_hbm.at[pl.ds(c * N + NS * ROWS_PT, ROWS_TAIL)])


_SC_AGG_CACHE = []


def _sc_agg(*args):
    if not _SC_AGG_CACHE:
        _SC_AGG_CACHE.append(_build_sc_agg())
    return _SC_AGG_CACHE[0](*args)


def _build_sc_agg():
    return functools.partial(
        pl.kernel,
        out_type=jax.ShapeDtypeStruct((NC * N, DH), jnp.float32),
        mesh=plsc.VectorSubcoreMesh(core_axis_name="c", subcore_axis_name="s"),
        compiler_params=pltpu.CompilerParams(needs_layout_passes=False),
        scratch_types=[
            [pltpu.VMEM((CHUNK,), jnp.int32)] * 2,
            [pltpu.VMEM((CHUNK,), jnp.int32)] * 2,
            [pltpu.VMEM((CHUNK,), jnp.int32)] * 2,
            pltpu.VMEM((TAIL,), jnp.int32),
            [pltpu.VMEM((CHUNK, DH), jnp.int32)] * 2,
            [pltpu.VMEM((CHUNK, DH), jnp.int32)] * 2,
            [pltpu.VMEM((CHUNK, DH), jnp.int32)] * 2,
            [pltpu.VMEM((CHUNK, DH // 2), jnp.int32)] * 2,
            pltpu.VMEM((CHUNK, DH), jnp.float32),
            pltpu.VMEM_SHARED((N, DH), jnp.float32),
            [pltpu.SemaphoreType.DMA] * 2,
            [pltpu.SemaphoreType.DMA] * 2,
        ],
    )(_sc_body)


def _stats_body(agg_ref, sum_ref, sq_ref):
    a = agg_ref[...]
    sum_ref[...] = jnp.sum(a, axis=0).reshape(1, 1, DH)
    sq_ref[...] = jnp.sum(a * a, axis=0).reshape(1, 1, DH)


def _norm_body(agg_ref, sums_ref, sqs_ref, gamma_ref, beta_ref, o_ref):
    mean = jnp.sum(sums_ref[...], axis=1) / N
    var = jnp.sum(sqs_ref[...], axis=1) / N - mean * mean
    inv = lax.rsqrt(var + 1e-5)
    x = agg_ref[...]
    o_ref[...] = jnp.maximum(gamma_ref[...] * (x - mean) * inv + beta_ref[...], 0.0)


def _tc_bn_relu(agg2, gamma, beta):
    BLK = 1000
    nblk = N // BLK
    grid = (NC * nblk,)
    sums, sqs = pl.pallas_call(
        _stats_body,
        grid=grid,
        in_specs=[pl.BlockSpec((BLK, DH), lambda i: (i, 0))],
        out_specs=[pl.BlockSpec((1, 1, DH), lambda i: (i, 0, 0))] * 2,
        out_shape=[jax.ShapeDtypeStruct((NC * nblk, 1, DH), jnp.float32)] * 2,
    )(agg2)
    sums = sums.reshape(NC, nblk, DH)
    sqs = sqs.reshape(NC, nblk, DH)
    return pl.pallas_call(
        _norm_body,
        grid=(NC, nblk),
        in_specs=[
            pl.BlockSpec((BLK, DH), lambda c, i: (c * nblk + i, 0)),
            pl.BlockSpec((1, nblk, DH), lambda c, i: (c, 0, 0)),
            pl.BlockSpec((1, nblk, DH), lambda c, i: (c, 0, 0)),
            pl.BlockSpec((1, DH), lambda c, i: (0, c)),
            pl.BlockSpec((1, DH), lambda c, i: (0, c)),
        ],
        out_specs=pl.BlockSpec((BLK, DH), lambda c, i: (i, c)),
        out_shape=jax.ShapeDtypeStruct((N, D), jnp.float32),
    )(agg2, sums, sqs, gamma[None, :], beta[None, :])


def _pack_words(t_bf16):
    r, ch = t_bf16.shape
    return lax.bitcast_convert_type(t_bf16.reshape(r, ch // 2, 2), jnp.int32)


def _layer(x, ef, src, dst, W_k, b_k, W_q, b_q, W_v, b_v, W_s, b_o, gamma, beta):
    k2, q2, v2, skip2 = _tc_proj(x, W_k, b_k, W_q, b_q, W_v, b_v, W_s, b_o)
    agg2 = _sc_agg(_pack_words(k2), _pack_words(q2), _pack_words(v2), ef,
                   src, dst, skip2)
    return _tc_bn_relu(agg2, gamma, beta)


def kernel(x, edge_index, x_edge,
           W_k1, b_k1, W_q1, b_q1, W_v1, b_v1, W_e1, b_e1, W_s1, b_o1, gamma1, beta1,
           W_k2, b_k2, W_q2, b_q2, W_v2, b_v2, W_e2, b_e2, W_s2, b_o2, gamma2, beta2):
    src = edge_index[0]
    dst = edge_index[1]
    ef1, ef2 = _tc_edge(x_edge, W_e1[:, _PERM], b_e1[_PERM], W_e2[:, _PERM], b_e2[_PERM])
    ef1 = _pack_words(ef1)
    ef2 = _pack_words(ef2)
    h = _layer(x, ef1, src, dst, W_k1[:, _PERM], b_k1[_PERM], W_q1[:, _PERM], b_q1[_PERM],
               W_v1[:, _PERM], b_v1[_PERM], W_s1, b_o1, gamma1, beta1)
    h = _layer(h, ef2, src, dst, W_k2[:, _PERM], b_k2[_PERM], W_q2[:, _PERM], b_q2[_PERM],
               W_v2[:, _PERM], b_v2[_PERM], W_s2, b_o2, gamma2, beta2)
    return h


# speedup vs baseline: 3.6173x; 3.6173x over previous
import functools

import jax
import jax.numpy as jnp
from jax import lax
from jax.experimental import pallas as pl
from jax.experimental.pallas import tpu as pltpu
from jax.experimental.pallas import tpu_sc as plsc

N = 10000
E = 160000
D = 256
DH = 128
NC = 2
NS = 16
L = 16

CHUNK = 48
EPT = E // NS
NCHUNKS = (EPT - 16) // CHUNK
TAIL = EPT - NCHUNKS * CHUNK
ROWS_PT = 624
ROWS_TAIL = N - ROWS_PT * NS


def _proj_body(x_ref, wk_ref, wq_ref, wv_ref, ws_ref, bk_ref, bq_ref, bv_ref,
               bo_ref, k_ref, q_ref, v_ref, s_ref):
    x = x_ref[...]
    k_ref[...] = jnp.dot(x, wk_ref[...], preferred_element_type=jnp.float32) + bk_ref[...]
    q_ref[...] = jnp.dot(x, wq_ref[...], preferred_element_type=jnp.float32) + bq_ref[...]
    v_ref[...] = jnp.dot(x, wv_ref[...], preferred_element_type=jnp.float32) + bv_ref[...]
    s_ref[...] = jnp.dot(x, ws_ref[...], preferred_element_type=jnp.float32) + bo_ref[...]


def _tc_proj(x, W_k, b_k, W_q, b_q, W_v, b_v, W_s, b_o):
    BLK = 1000
    grid = (N // BLK, NC)
    w_spec = pl.BlockSpec((D, DH), lambda i, j: (0, j))
    b_spec = pl.BlockSpec((1, DH), lambda i, j: (0, j))
    out_spec = pl.BlockSpec((BLK, DH), lambda i, j: (j * (N // BLK) + i, 0))
    out_shape = jax.ShapeDtypeStruct((NC * N, DH), jnp.float32)
    return pl.pallas_call(
        _proj_body,
        grid=grid,
        in_specs=[pl.BlockSpec((BLK, D), lambda i, j: (i, 0))] + [w_spec] * 4 + [b_spec] * 4,
        out_specs=[out_spec] * 4,
        out_shape=[out_shape] * 4,
    )(x, W_k, W_q, W_v, W_s, b_k[None, :], b_q[None, :], b_v[None, :], b_o[None, :])


def _edge_body(xe_ref, w1_ref, w2_ref, b1_ref, b2_ref, e1_ref, e2_ref):
    xe = xe_ref[...]
    e1_ref[...] = jnp.dot(xe, w1_ref[...], preferred_element_type=jnp.float32) + b1_ref[...]
    e2_ref[...] = jnp.dot(xe, w2_ref[...], preferred_element_type=jnp.float32) + b2_ref[...]


def _tc_edge(x_edge, W_e1, b_e1, W_e2, b_e2):
    BLK = 4000
    DE = x_edge.shape[1]
    grid = (E // BLK, NC)
    w_spec = pl.BlockSpec((DE, DH), lambda i, j: (0, j))
    b_spec = pl.BlockSpec((1, DH), lambda i, j: (0, j))
    out_spec = pl.BlockSpec((BLK, DH), lambda i, j: (j * (E // BLK) + i, 0))
    out_shape = jax.ShapeDtypeStruct((NC * E, DH), jnp.float32)
    return pl.pallas_call(
        _edge_body,
        grid=grid,
        in_specs=[pl.BlockSpec((BLK, DE), lambda i, j: (i, 0))] + [w_spec] * 2 + [b_spec] * 2,
        out_specs=[out_spec] * 2,
        out_shape=[out_shape] * 2,
    )(x_edge, W_e1, W_e2, b_e1[None, :], b_e2[None, :])


def _sc_body(k_hbm, q_hbm, v_hbm, ef_hbm, src_hbm, dst_hbm, skip_hbm, out_hbm,
             sidx, didx, doff, dscat, tdsc,
             buf_k, buf_q, buf_v, buf_e,
             agg, sem_i, sem_g):
    c = lax.axis_index("c")
    s = lax.axis_index("s")

    row0 = s * ROWS_PT
    pltpu.sync_copy(skip_hbm.at[pl.ds(c * N + row0, ROWS_PT)],
                    agg.at[pl.ds(row0, ROWS_PT)])

    @pl.when(s == NS - 1)
    def _():
        pltpu.sync_copy(skip_hbm.at[pl.ds(c * N + NS * ROWS_PT, ROWS_TAIL)],
                        agg.at[pl.ds(NS * ROWS_PT, ROWS_TAIL)])

    plsc.subcore_barrier()

    ebase = s * EPT
    coff = c * N

    def clamp(i):
        return jnp.minimum(i, NCHUNKS - 1)

    def stage_idx(i, b):
        base = ebase + clamp(i) * CHUNK
        pltpu.async_copy(src_hbm.at[pl.ds(base, CHUNK)], sidx[b], sem_i[b])
        pltpu.async_copy(dst_hbm.at[pl.ds(base, CHUNK)], didx[b], sem_i[b])

    def wait_idx(b):
        pltpu.make_async_copy(src_hbm.at[pl.ds(ebase, CHUNK)], sidx[b], sem_i[b]).wait()
        pltpu.make_async_copy(dst_hbm.at[pl.ds(ebase, CHUNK)], didx[b], sem_i[b]).wait()

    def fire_gathers(i, b):
        for j in range(CHUNK // L):
            sl = pl.ds(j * L, L)
            d = didx[b][sl]
            dscat[b][sl] = d
            doff[b][sl] = d + coff
            sidx[b][sl] = sidx[b][sl] + coff
        base = ebase + clamp(i) * CHUNK
        pltpu.async_copy(k_hbm.at[doff[b]], buf_k[b], sem_g[b])
        pltpu.async_copy(q_hbm.at[sidx[b]], buf_q[b], sem_g[b])
        pltpu.async_copy(v_hbm.at[sidx[b]], buf_v[b], sem_g[b])
        pltpu.async_copy(ef_hbm.at[pl.ds(c * E + base, CHUNK)], buf_e[b], sem_g[b])

    def wait_gathers(b):
        pltpu.make_async_copy(k_hbm.at[doff[b]], buf_k[b], sem_g[b]).wait()
        pltpu.make_async_copy(q_hbm.at[sidx[b]], buf_q[b], sem_g[b]).wait()
        pltpu.make_async_copy(v_hbm.at[sidx[b]], buf_v[b], sem_g[b]).wait()
        pltpu.make_async_copy(ef_hbm.at[pl.ds(c * E, CHUNK)], buf_e[b], sem_g[b]).wait()

    def compute(b, nedge):
        def edge_body(r, carry2):
            for j in range(DH // L):
                sl = pl.ds(j * L, L)
                z = buf_k[b][r, sl] + buf_q[b][r, sl] + buf_e[b][r, sl]
                g = 1.0 / (1.0 + jnp.exp(-z))
                buf_k[b][r, sl] = g * buf_v[b][r, sl]
            return carry2

        lax.fori_loop(0, nedge, edge_body, 0)

    stage_idx(0, 0)
    stage_idx(1, 1)
    wait_idx(0)
    fire_gathers(0, 0)

    def pipe_body(ig, carry):
        for b in (0, 1):
            i = 2 * ig + b
            wait_gathers(b)
            stage_idx(i + 2, b)
            wait_idx(1 - b)
            fire_gathers(i + 1, 1 - b)
            compute(b, CHUNK)
            pltpu.sync_copy(buf_k[b], agg.at[dscat[b]], add=True)
        return carry

    lax.fori_loop(0, NCHUNKS // 2, pipe_body, 0)

    wait_gathers(0)
    wait_idx(1)

    tbase = ebase + NCHUNKS * CHUNK
    pltpu.sync_copy(src_hbm.at[pl.ds(tbase, TAIL)], sidx[0].at[pl.ds(0, TAIL)])
    pltpu.sync_copy(dst_hbm.at[pl.ds(tbase, TAIL)], didx[0].at[pl.ds(0, TAIL)])
    d = didx[0][pl.ds(0, L)]
    tdsc[...] = d
    doff[1][pl.ds(0, L)] = d + coff
    sidx[1][pl.ds(0, L)] = sidx[0][pl.ds(0, L)] + coff
    tdoff = doff[1].at[pl.ds(0, TAIL)]
    tsoff = sidx[1].at[pl.ds(0, TAIL)]
    cp_k = pltpu.async_copy(k_hbm.at[tdoff], buf_k[0].at[pl.ds(0, TAIL)], sem_g[0])
    cp_q = pltpu.async_copy(q_hbm.at[tsoff], buf_q[0].at[pl.ds(0, TAIL)], sem_g[0])
    cp_v = pltpu.async_copy(v_hbm.at[tsoff], buf_v[0].at[pl.ds(0, TAIL)], sem_g[0])
    cp_e = pltpu.async_copy(ef_hbm.at[pl.ds(c * E + tbase, TAIL)],
                            buf_e[0].at[pl.ds(0, TAIL)], sem_g[0])
    cp_k.wait()
    cp_q.wait()
    cp_v.wait()
    cp_e.wait()
    compute(0, TAIL)
    pltpu.sync_copy(buf_k[0].at[pl.ds(0, TAIL)], agg.at[tdsc], add=True)

    plsc.subcore_barrier()

    pltpu.sync_copy(agg.at[pl.ds(row0, ROWS_PT)],
                    out_hbm.at[pl.ds(c * N + row0, ROWS_PT)])

    @pl.when(s == NS - 1)
    def _():
        pltpu.sync_copy(agg.at[pl.ds(NS * ROWS_PT, ROWS_TAIL)],
                        out_hbm.at[pl.ds(c * N + NS * ROWS_PT, ROWS_TAIL)])


_SC_AGG_CACHE = []


def _sc_agg(*args):
    if not _SC_AGG_CACHE:
        _SC_AGG_CACHE.append(_build_sc_agg())
    return _SC_AGG_CACHE[0](*args)


def _build_sc_agg():
    return functools.partial(
        pl.kernel,
        out_type=jax.ShapeDtypeStruct((NC * N, DH), jnp.float32),
        mesh=plsc.VectorSubcoreMesh(core_axis_name="c", subcore_axis_name="s"),
        scratch_types=[
            [pltpu.VMEM((CHUNK,), jnp.int32)] * 2,
            [pltpu.VMEM((CHUNK,), jnp.int32)] * 2,
            [pltpu.VMEM((CHUNK,), jnp.int32)] * 2,
            [pltpu.VMEM((CHUNK,), jnp.int32)] * 2,
            pltpu.VMEM((TAIL,), jnp.int32),
            [pltpu.VMEM((CHUNK, DH), jnp.float32)] * 2,
            [pltpu.VMEM((CHUNK, DH), jnp.float32)] * 2,
            [pltpu.VMEM((CHUNK, DH), jnp.float32)] * 2,
            [pltpu.VMEM((CHUNK, DH), jnp.float32)] * 2,
            pltpu.VMEM_SHARED((N, DH), jnp.float32),
            [pltpu.SemaphoreType.DMA] * 2,
            [pltpu.SemaphoreType.DMA] * 2,
        ],
    )(_sc_body)


def _stats_body(agg_ref, sum_ref, sq_ref):
    a = agg_ref[...]
    sum_ref[...] = jnp.sum(a, axis=0).reshape(1, 1, DH)
    sq_ref[...] = jnp.sum(a * a, axis=0).reshape(1, 1, DH)


def _norm_body(agg_ref, sums_ref, sqs_ref, gamma_ref, beta_ref, o_ref):
    mean = jnp.sum(sums_ref[...], axis=1) / N
    var = jnp.sum(sqs_ref[...], axis=1) / N - mean * mean
    inv = lax.rsqrt(var + 1e-5)
    x = agg_ref[...]
    o_ref[...] = jnp.maximum(gamma_ref[...] * (x - mean) * inv + beta_ref[...], 0.0)


def _tc_bn_relu(agg2, gamma, beta):
    BLK = 1000
    nblk = N // BLK
    grid = (NC * nblk,)
    sums, sqs = pl.pallas_call(
        _stats_body,
        grid=grid,
        in_specs=[pl.BlockSpec((BLK, DH), lambda i: (i, 0))],
        out_specs=[pl.BlockSpec((1, 1, DH), lambda i: (i, 0, 0))] * 2,
        out_shape=[jax.ShapeDtypeStruct((NC * nblk, 1, DH), jnp.float32)] * 2,
    )(agg2)
    sums = sums.reshape(NC, nblk, DH)
    sqs = sqs.reshape(NC, nblk, DH)
    return pl.pallas_call(
        _norm_body,
        grid=(NC, nblk),
        in_specs=[
            pl.BlockSpec((BLK, DH), lambda c, i: (c * nblk + i, 0)),
            pl.BlockSpec((1, nblk, DH), lambda c, i: (c, 0, 0)),
            pl.BlockSpec((1, nblk, DH), lambda c, i: (c, 0, 0)),
            pl.BlockSpec((1, DH), lambda c, i: (0, c)),
            pl.BlockSpec((1, DH), lambda c, i: (0, c)),
        ],
        out_specs=pl.BlockSpec((BLK, DH), lambda c, i: (i, c)),
        out_shape=jax.ShapeDtypeStruct((N, D), jnp.float32),
    )(agg2, sums, sqs, gamma[None, :], beta[None, :])


def _layer(x, ef, src, dst, W_k, b_k, W_q, b_q, W_v, b_v, W_s, b_o, gamma, beta):
    k2, q2, v2, skip2 = _tc_proj(x, W_k, b_k, W_q, b_q, W_v, b_v, W_s, b_o)
    agg2 = _sc_agg(k2, q2, v2, ef, src, dst, skip2)
    return _tc_bn_relu(agg2, gamma, beta)


def kernel(x, edge_index, x_edge,
           W_k1, b_k1, W_q1, b_q1, W_v1, b_v1, W_e1, b_e1, W_s1, b_o1, gamma1, beta1,
           W_k2, b_k2, W_q2, b_q2, W_v2, b_v2, W_e2, b_e2, W_s2, b_o2, gamma2, beta2):
    src = edge_index[0]
    dst = edge_index[1]
    ef1, ef2 = _tc_edge(x_edge, W_e1, b_e1, W_e2, b_e2)
    h = _layer(x, ef1, src, dst, W_k1, b_k1, W_q1, b_q1, W_v1, b_v1, W_s1, b_o1, gamma1, beta1)
    h = _layer(h, ef2, src, dst, W_k2, b_k2, W_q2, b_q2, W_v2, b_v2, W_s2, b_o2, gamma2, beta2)
    return h
